# Initial kernel scaffold; baseline (speedup 1.0000x reference)
import jax
import jax.numpy as jnp
from jax.experimental import pallas as pl


def kernel(z_protein, z_class, edge_label_index, W, b):
    raise NotImplementedError("write your pallas kernel here")


# speedup vs baseline: 2.6628x; 2.6628x over previous
import functools

import jax
import jax.numpy as jnp
from jax import lax
from jax.experimental import pallas as pl
from jax.experimental.pallas import tpu as pltpu
from jax.experimental.pallas import tpu_sc as plsc

_NC = 2
_NS = 16
_NW = _NC * _NS
_L = 16
_K = 128
_D = 128
_G = _D // _L


def _sc_link_predict(zp, zc, src2d, tgt2d, w_flat, b_vec, n_chunks):
    n_iter = (n_chunks + _NW - 1) // _NW

    mesh = plsc.VectorSubcoreMesh(core_axis_name="c", subcore_axis_name="s")

    @functools.partial(
        pl.kernel,
        mesh=mesh,
        out_type=jax.ShapeDtypeStruct((n_chunks, _K), jnp.float32),
        compiler_params=pltpu.CompilerParams(needs_layout_passes=False),
        scratch_types=[
            pltpu.VMEM((_K,), jnp.int32),
            pltpu.VMEM((_K,), jnp.int32),
            pltpu.VMEM((_K, _D), jnp.float32),
            pltpu.VMEM((_K, _D), jnp.float32),
            pltpu.VMEM((_D,), jnp.float32),
            pltpu.VMEM((_L,), jnp.float32),
            pltpu.VMEM((_K,), jnp.float32),
            pltpu.SemaphoreType.DMA,
        ],
    )
    def body(zp_r, zc_r, src_r, tgt_r, w_r, b_r, out_r,
             idx_s, idx_t, rows_s, rows_t, w_v, b_v, out_v, sem):
        wid = lax.axis_index("s") * _NC + lax.axis_index("c")

        pltpu.sync_copy(w_r, w_v)
        pltpu.sync_copy(b_r, b_v)
        bvec = b_v[...]
        lane = lax.iota(jnp.int32, _L)
        wvals = [w_v[pl.ds(g * _L, _L)] for g in range(_G)]

        def outer(i, carry):
            c = wid + i * _NW

            @pl.when(c < n_chunks)
            def _():
                pltpu.sync_copy(src_r.at[c], idx_s)
                pltpu.sync_copy(tgt_r.at[c], idx_t)
                cp_s = pltpu.async_copy(zp_r.at[idx_s], rows_s, sem)
                cp_t = pltpu.async_copy(zc_r.at[idx_t], rows_t, sem)
                cp_s.wait()
                cp_t.wait()

                def egroup_step(eg, ecarry):
                    tot = jnp.zeros((_L,), jnp.float32)
                    for j in range(_L):
                        e = eg * _L + j
                        acc = rows_s[e, pl.ds(0, _L)] * rows_t[e, pl.ds(0, _L)] * wvals[0]
                        for g in range(1, _G):
                            sl = pl.ds(g * _L, _L)
                            acc = acc + rows_s[e, sl] * rows_t[e, sl] * wvals[g]
                        s = jnp.sum(acc)
                        tot = jnp.where(lane == j, s, tot)
                    x = tot + bvec
                    out_v[pl.ds(eg * _L, _L)] = 1.0 / (1.0 + jnp.exp(-x))
                    return ecarry

                lax.fori_loop(0, _K // _L, egroup_step, 0)

                pltpu.sync_copy(out_v, out_r.at[c])

            return carry

        lax.fori_loop(0, n_iter, outer, 0)

    return body(zp, zc, src2d, tgt2d, w_flat, b_vec)


def kernel(z_protein, z_class, edge_label_index, W, b):
    n_edges = edge_label_index.shape[1]
    n_chunks = n_edges // _K
    src2d = edge_label_index[0].astype(jnp.int32).reshape(n_chunks, _K)
    tgt2d = edge_label_index[1].astype(jnp.int32).reshape(n_chunks, _K)
    w_flat = W.reshape(_D).astype(jnp.float32)
    b_vec = jnp.broadcast_to(b.reshape(()), (_L,)).astype(jnp.float32)
    out2d = _sc_link_predict(z_protein, z_class, src2d, tgt2d, w_flat, b_vec,
                             n_chunks)
    return out2d.reshape(n_edges)


# speedup vs baseline: 3.0400x; 1.1416x over previous
import functools

import jax
import jax.numpy as jnp
from jax import lax
from jax.experimental import pallas as pl
from jax.experimental.pallas import tpu as pltpu
from jax.experimental.pallas import tpu_sc as plsc

_NC = 2
_NS = 16
_NW = _NC * _NS
_L = 16
_K = 128
_D = 128
_G = _D // _L
_CPW = 80


def _sc_link_predict(zp, zc, src2d, tgt2d, w_flat, b_vec):
    n_chunks = src2d.shape[0]

    mesh = plsc.VectorSubcoreMesh(core_axis_name="c", subcore_axis_name="s")

    @functools.partial(
        pl.kernel,
        mesh=mesh,
        out_type=jax.ShapeDtypeStruct((n_chunks, _K), jnp.float32),
        compiler_params=pltpu.CompilerParams(needs_layout_passes=False),
        scratch_types=[
            pltpu.VMEM((_CPW, _K), jnp.int32),
            pltpu.VMEM((_CPW, _K), jnp.int32),
            pltpu.VMEM((_K, _D), jnp.float32),
            pltpu.VMEM((_K, _D), jnp.float32),
            pltpu.VMEM((_K, _D), jnp.float32),
            pltpu.VMEM((_K, _D), jnp.float32),
            pltpu.VMEM((_CPW, _K), jnp.float32),
            pltpu.VMEM((_D,), jnp.float32),
            pltpu.VMEM((_L,), jnp.float32),
            pltpu.SemaphoreType.DMA,
            pltpu.SemaphoreType.DMA,
        ],
    )
    def body(zp_r, zc_r, src_r, tgt_r, w_r, b_r, out_r,
             idx_s_sl, idx_t_sl, rs0, rs1, rt0, rt1, out_sl, w_v, b_v,
             sem0, sem1):
        wid = lax.axis_index("s") * _NC + lax.axis_index("c")
        start = wid * _CPW

        pltpu.sync_copy(w_r, w_v)
        pltpu.sync_copy(b_r, b_v)
        bvec = b_v[...]
        lane = lax.iota(jnp.int32, _L)
        wvals = [w_v[pl.ds(g * _L, _L)] for g in range(_G)]

        pltpu.sync_copy(src_r.at[pl.ds(start, _CPW)], idx_s_sl)
        pltpu.sync_copy(tgt_r.at[pl.ds(start, _CPW)], idx_t_sl)

        bufs = ((rs0, rt0, sem0), (rs1, rt1, sem1))

        def issue(c, b):
            rs, rt, sem = bufs[b]
            pltpu.async_copy(zp_r.at[idx_s_sl.at[c]], rs, sem)
            pltpu.async_copy(zc_r.at[idx_t_sl.at[c]], rt, sem)

        def wait(c, b):
            rs, rt, sem = bufs[b]
            pltpu.make_async_copy(zp_r.at[idx_s_sl.at[c]], rs, sem).wait()
            pltpu.make_async_copy(zc_r.at[idx_t_sl.at[c]], rt, sem).wait()

        def compute(c, b):
            rs, rt, _ = bufs[b]

            def egroup_step(eg, ecarry):
                tot = jnp.zeros((_L,), jnp.float32)
                for j in range(_L):
                    e = eg * _L + j
                    acc = rs[e, pl.ds(0, _L)] * rt[e, pl.ds(0, _L)] * wvals[0]
                    for g in range(1, _G):
                        sl = pl.ds(g * _L, _L)
                        acc = acc + rs[e, sl] * rt[e, sl] * wvals[g]
                    s = jnp.sum(acc)
                    tot = jnp.where(lane == j, s, tot)
                x = tot + bvec
                out_sl[c, pl.ds(eg * _L, _L)] = 1.0 / (1.0 + jnp.exp(-x))
                return ecarry

            lax.fori_loop(0, _K // _L, egroup_step, 0)

        issue(0, 0)

        def outer(i, carry):
            for b in range(2):
                c = i * 2 + b

                @pl.when(c + 1 < _CPW)
                def _():
                    issue(c + 1, 1 - b)

                wait(c, b)
                compute(c, b)
            return carry

        lax.fori_loop(0, _CPW // 2, outer, 0)

        pltpu.sync_copy(out_sl, out_r.at[pl.ds(start, _CPW)])

    return body(zp, zc, src2d, tgt2d, w_flat, b_vec)


def kernel(z_protein, z_class, edge_label_index, W, b):
    n_edges = edge_label_index.shape[1]
    n_pad = _NW * _CPW * _K
    src = edge_label_index[0].astype(jnp.int32)
    tgt = edge_label_index[1].astype(jnp.int32)
    pad = jnp.zeros((n_pad - n_edges,), jnp.int32)
    src2d = jnp.concatenate([src, pad]).reshape(n_pad // _K, _K)
    tgt2d = jnp.concatenate([tgt, pad]).reshape(n_pad // _K, _K)
    w_flat = W.reshape(_D).astype(jnp.float32)
    b_vec = jnp.broadcast_to(b.reshape(()), (_L,)).astype(jnp.float32)
    out2d = _sc_link_predict(z_protein, z_class, src2d, tgt2d, w_flat, b_vec)
    return out2d.reshape(n_pad)[:n_edges]


# speedup vs baseline: 3.1409x; 1.0332x over previous
import functools

import jax
import jax.numpy as jnp
from jax import lax
from jax.experimental import pallas as pl
from jax.experimental.pallas import tpu as pltpu
from jax.experimental.pallas import tpu_sc as plsc

_NC = 2
_NS = 16
_NW = _NC * _NS
_L = 16
_K = 128
_D = 128
_G = _D // _L
_CPW = 80


def _sc_link_predict(zp, zc, src2d, tgt2d, w_flat, b_vec):
    n_chunks = src2d.shape[0]

    mesh = plsc.VectorSubcoreMesh(core_axis_name="c", subcore_axis_name="s")

    @functools.partial(
        pl.kernel,
        mesh=mesh,
        out_type=jax.ShapeDtypeStruct((n_chunks, _K), jnp.float32),
        compiler_params=pltpu.CompilerParams(needs_layout_passes=False),
        scratch_types=[
            pltpu.VMEM((_CPW, _K), jnp.int32),
            pltpu.VMEM((_CPW, _K), jnp.int32),
            pltpu.VMEM((_K, _D), jnp.float32),
            pltpu.VMEM((_K, _D), jnp.float32),
            pltpu.VMEM((_K, _D), jnp.float32),
            pltpu.VMEM((_K, _D), jnp.float32),
            pltpu.VMEM((_CPW, _K), jnp.float32),
            pltpu.VMEM((_D,), jnp.float32),
            pltpu.VMEM((_L,), jnp.float32),
            pltpu.SemaphoreType.DMA,
            pltpu.SemaphoreType.DMA,
        ],
    )
    def body(zp_r, zc_r, src_r, tgt_r, w_r, b_r, out_r,
             idx_s_sl, idx_t_sl, rs0, rs1, rt0, rt1, out_sl, w_v, b_v,
             sem0, sem1):
        wid = lax.axis_index("s") * _NC + lax.axis_index("c")
        start = wid * _CPW

        pltpu.sync_copy(w_r, w_v)
        pltpu.sync_copy(b_r, b_v)
        bvec = b_v[...]
        lane = lax.iota(jnp.int32, _L)
        wvals = [w_v[pl.ds(g * _L, _L)] for g in range(_G)]

        pltpu.sync_copy(src_r.at[pl.ds(start, _CPW)], idx_s_sl)
        pltpu.sync_copy(tgt_r.at[pl.ds(start, _CPW)], idx_t_sl)

        bufs = ((rs0, rt0, sem0), (rs1, rt1, sem1))

        def issue(c, b):
            rs, rt, sem = bufs[b]
            pltpu.async_copy(zp_r.at[idx_s_sl.at[c]], rs, sem)
            pltpu.async_copy(zc_r.at[idx_t_sl.at[c]], rt, sem)

        def wait(c, b):
            rs, rt, sem = bufs[b]
            pltpu.make_async_copy(zp_r.at[idx_s_sl.at[c]], rs, sem).wait()
            pltpu.make_async_copy(zc_r.at[idx_t_sl.at[c]], rt, sem).wait()

        def compute(c, b):
            rs, rt, _ = bufs[b]

            def egroup_step(eg, ecarry):
                def sub(jj, tot):
                    for k in range(4):
                        j = jj * 4 + k
                        e = eg * _L + j
                        q = [rs[e, pl.ds(g * _L, _L)]
                             * rt[e, pl.ds(g * _L, _L)] * wvals[g]
                             for g in range(_G)]
                        acc = ((q[0] + q[1]) + (q[2] + q[3])) \
                            + ((q[4] + q[5]) + (q[6] + q[7]))
                        s = jnp.sum(acc)
                        tot = jnp.where(lane == j, s, tot)
                    return tot

                tot = lax.fori_loop(0, _L // 4, sub,
                                    jnp.zeros((_L,), jnp.float32))
                x = tot + bvec
                out_sl[c, pl.ds(eg * _L, _L)] = 1.0 / (1.0 + jnp.exp(-x))
                return ecarry

            lax.fori_loop(0, _K // _L, egroup_step, 0)

        issue(0, 0)

        def outer(i, carry):
            for b in range(2):
                c = i * 2 + b

                @pl.when(c + 1 < _CPW)
                def _():
                    issue(c + 1, 1 - b)

                wait(c, b)
                compute(c, b)
            return carry

        lax.fori_loop(0, _CPW // 2, outer, 0)

        pltpu.sync_copy(out_sl, out_r.at[pl.ds(start, _CPW)])

    return body(zp, zc, src2d, tgt2d, w_flat, b_vec)


def kernel(z_protein, z_class, edge_label_index, W, b):
    n_edges = edge_label_index.shape[1]
    n_pad = _NW * _CPW * _K
    src = edge_label_index[0].astype(jnp.int32)
    tgt = edge_label_index[1].astype(jnp.int32)
    pad = jnp.zeros((n_pad - n_edges,), jnp.int32)
    src2d = jnp.concatenate([src, pad]).reshape(n_pad // _K, _K)
    tgt2d = jnp.concatenate([tgt, pad]).reshape(n_pad // _K, _K)
    w_flat = W.reshape(_D).astype(jnp.float32)
    b_vec = jnp.broadcast_to(b.reshape(()), (_L,)).astype(jnp.float32)
    out2d = _sc_link_predict(z_protein, z_class, src2d, tgt2d, w_flat, b_vec)
    return out2d.reshape(n_pad)[:n_edges]
